# baseline (speedup 1.0000x reference)
import functools

import jax
import jax.numpy as jnp
from jax import lax
from jax.experimental import pallas as pl
from jax.experimental.pallas import tpu as pltpu
from jax.experimental.pallas import tpu_sc as plsc

NUM_CLS = 151
FILL_V = 1000.0
LANES = 16
STRIP = 128


def _sc_workers():
    try:
        info = plsc.get_sparse_core_info()
        return info.num_cores, info.num_subcores
    except Exception:
        return 2, 16


def _onehot_call(n_rows: int):
    NC, NS = _sc_workers()
    NW = NC * NS
    C = NUM_CLS
    assert n_rows % 8 == 0 and n_rows >= STRIP
    c_pad = -(-C // 8) * 8
    n_tiles = c_pad // 8
    n_strips = -(-n_rows // STRIP)
    max_strips = -(-n_strips // NW)
    kF = max_strips - 1
    last_w = n_rows - (n_strips - 1) * STRIP
    i0_last = (n_strips - 1) * STRIP
    shift = i0_last - (n_rows - STRIP)

    mesh = plsc.VectorSubcoreMesh(core_axis_name="c", subcore_axis_name="s")

    scratch = [pltpu.VMEM((STRIP,), jnp.int32) for _ in range(max_strips)]
    scratch += [pltpu.VMEM((c_pad, STRIP), jnp.float32) for _ in range(2)]
    scratch += [pltpu.SemaphoreType.DMA for _ in range(max_strips + 2)]

    @functools.partial(
        pl.kernel,
        out_type=jax.ShapeDtypeStruct((C, n_rows), jnp.float32),
        mesh=mesh,
        scratch_types=scratch,
        compiler_params=pltpu.CompilerParams(needs_layout_passes=False),
    )
    def onehot_kernel(labels_hbm, out_hbm, *scr):
        labs = scr[:max_strips]
        bufs = scr[max_strips:max_strips + 2]
        sem_labs = scr[max_strips + 2:2 * max_strips + 2]
        sems = scr[2 * max_strips + 2:2 * max_strips + 4]
        wid = lax.axis_index("s") * NC + lax.axis_index("c")

        neg = jnp.full((LANES,), -FILL_V, dtype=jnp.float32)
        pos = jnp.full((LANES,), FILL_V, dtype=jnp.float32)
        lane = lax.iota(jnp.int32, LANES)

        lab_descs = []
        for k in range(max_strips):
            s = wid + k * NW
            i0_lab = jnp.minimum(s * STRIP, n_rows - STRIP)
            d = pltpu.make_async_copy(
                labels_hbm.at[pl.ds(i0_lab, STRIP)], labs[k], sem_labs[k])
            d.start()
            lab_descs.append(d)

        def fill_body(tt, _):
            for b in range(2):
                for r8 in range(8):
                    for o in range(0, STRIP, LANES):
                        bufs[b][tt * 8 + r8, pl.ds(o, LANES)] = neg
            return _

        lax.fori_loop(0, c_pad // 8, fill_body, None)

        for d in lab_descs:
            d.wait()

        def scatter(buf, labref, val):
            for j in range(STRIP // LANES):
                lab = labref[pl.ds(j * LANES, LANES)]
                plsc.store_scatter(buf, [lab, (j * LANES) + lane], val)

        def fire(buf, i0, sem):
            descs = []
            for tr in range(n_tiles):
                rows = min(8, C - tr * 8)
                d = pltpu.make_async_copy(
                    buf.at[pl.ds(tr * 8, rows)],
                    out_hbm.at[pl.ds(tr * 8, rows), pl.ds(i0, STRIP)],
                    sem,
                )
                d.start()
                descs.append(d)
            return descs

        out_descs = []
        for k in range(kF):
            b = k % 2
            if k >= 2:
                for d in out_descs[k - 2]:
                    d.wait()
                scatter(bufs[b], labs[k - 2], neg)
            scatter(bufs[b], labs[k], pos)
            out_descs.append(fire(bufs[b], (wid + k * NW) * STRIP, sems[b]))

        bF = kF % 2
        sF = wid + kF * NW
        if kF >= 2:
            for d in out_descs[kF - 2]:
                d.wait()

            @pl.when(sF < n_strips)
            def _():
                scatter(bufs[bF], labs[kF - 2], neg)

        @pl.when(sF < n_strips - 1)
        def _():
            scatter(bufs[bF], labs[kF], pos)
            for d in fire(bufs[bF], sF * STRIP, sems[bF]):
                d.wait()

        @pl.when(sF == n_strips - 1)
        def _():
            for j in range(STRIP // LANES):
                lab = labs[kF][pl.ds(j * LANES, LANES)]
                il = (j * LANES) + lane - shift
                plsc.store_scatter(bufs[bF], [lab, jnp.maximum(il, 0)], pos,
                                   mask=il >= 0)
            i0p = sF * STRIP
            for tr in range(n_tiles):
                rows = min(8, C - tr * 8)
                pltpu.sync_copy(
                    bufs[bF].at[pl.ds(tr * 8, rows), pl.ds(0, last_w)],
                    out_hbm.at[pl.ds(tr * 8, rows), pl.ds(i0p, last_w)],
                )

        for k in range(max(0, kF - 1), kF):
            for d in out_descs[k]:
                d.wait()

    return onehot_kernel


def kernel(im_inds, obj_fmaps, obj_labels, rel_inds):
    n = obj_labels.shape[0]
    call = _onehot_call(n)
    return call(obj_labels).T


# speedup vs baseline: 1.0251x; 1.0251x over previous
import functools

import jax
import jax.numpy as jnp
from jax import lax
from jax.experimental import pallas as pl
from jax.experimental.pallas import tpu as pltpu
from jax.experimental.pallas import tpu_sc as plsc

NUM_CLS = 151
FILL_V = 1000.0
LANES = 16
STRIP = 128


def _sc_workers():
    try:
        info = plsc.get_sparse_core_info()
        return info.num_cores, info.num_subcores
    except Exception:
        return 2, 16


def _onehot_call(n_rows: int):
    NC, NS = _sc_workers()
    NW = NC * NS
    C = NUM_CLS
    assert n_rows % 8 == 0 and n_rows >= STRIP
    c_pad = -(-C // 8) * 8
    n_tiles = c_pad // 8
    n_strips = -(-n_rows // STRIP)
    max_strips = -(-n_strips // NW)
    kF = max_strips - 1
    last_w = n_rows - (n_strips - 1) * STRIP
    i0_last = (n_strips - 1) * STRIP
    shift = i0_last - (n_rows - STRIP)

    mesh = plsc.VectorSubcoreMesh(core_axis_name="c", subcore_axis_name="s")

    scratch = [pltpu.VMEM((STRIP,), jnp.int32) for _ in range(max_strips)]
    scratch += [pltpu.VMEM((c_pad, STRIP), jnp.float32) for _ in range(2)]
    scratch += [pltpu.SemaphoreType.DMA for _ in range(max_strips + 2)]

    @functools.partial(
        pl.kernel,
        out_type=jax.ShapeDtypeStruct((C, n_rows), jnp.float32),
        mesh=mesh,
        scratch_types=scratch,
        compiler_params=pltpu.CompilerParams(needs_layout_passes=False),
    )
    def onehot_kernel(labels_hbm, out_hbm, *scr):
        labs = scr[:max_strips]
        bufs = scr[max_strips:max_strips + 2]
        sem_labs = scr[max_strips + 2:2 * max_strips + 2]
        sems = scr[2 * max_strips + 2:2 * max_strips + 4]
        wid = lax.axis_index("s") * NC + lax.axis_index("c")

        neg = jnp.full((LANES,), -FILL_V, dtype=jnp.float32)
        pos = jnp.full((LANES,), FILL_V, dtype=jnp.float32)
        lane = lax.iota(jnp.int32, LANES)

        lab_descs = []
        for k in range(max_strips):
            s = wid + k * NW
            i0_lab = jnp.minimum(s * STRIP, n_rows - STRIP)
            d = pltpu.make_async_copy(
                labels_hbm.at[pl.ds(i0_lab, STRIP)], labs[k], sem_labs[k])
            d.start()
            lab_descs.append(d)

        def fill_body(rr, _):
            for b in range(2):
                for o in range(0, STRIP, LANES):
                    bufs[b][rr, pl.ds(o, LANES)] = neg
            return _

        lax.fori_loop(0, c_pad, fill_body, None)

        for d in lab_descs:
            d.wait()

        def scatter(buf, labref, val):
            for j in range(STRIP // LANES):
                lab = labref[pl.ds(j * LANES, LANES)]
                plsc.store_scatter(buf, [lab, (j * LANES) + lane], val)

        def fire(buf, i0, sem):
            descs = []
            for tr in range(n_tiles):
                rows = min(8, C - tr * 8)
                d = pltpu.make_async_copy(
                    buf.at[pl.ds(tr * 8, rows)],
                    out_hbm.at[pl.ds(tr * 8, rows), pl.ds(i0, STRIP)],
                    sem,
                )
                d.start()
                descs.append(d)
            return descs

        out_descs = []
        for k in range(kF):
            b = k % 2
            if k >= 2:
                for d in out_descs[k - 2]:
                    d.wait()
                scatter(bufs[b], labs[k - 2], neg)
            scatter(bufs[b], labs[k], pos)
            out_descs.append(fire(bufs[b], (wid + k * NW) * STRIP, sems[b]))

        bF = kF % 2
        sF = wid + kF * NW
        if kF >= 2:
            for d in out_descs[kF - 2]:
                d.wait()

            @pl.when(sF < n_strips)
            def _():
                scatter(bufs[bF], labs[kF - 2], neg)

        @pl.when(sF < n_strips - 1)
        def _():
            scatter(bufs[bF], labs[kF], pos)
            for d in fire(bufs[bF], sF * STRIP, sems[bF]):
                d.wait()

        @pl.when(sF == n_strips - 1)
        def _():
            for j in range(STRIP // LANES):
                lab = labs[kF][pl.ds(j * LANES, LANES)]
                il = (j * LANES) + lane - shift
                plsc.store_scatter(bufs[bF], [lab, jnp.maximum(il, 0)], pos,
                                   mask=il >= 0)
            i0p = sF * STRIP
            for tr in range(n_tiles):
                rows = min(8, C - tr * 8)
                pltpu.sync_copy(
                    bufs[bF].at[pl.ds(tr * 8, rows), pl.ds(0, last_w)],
                    out_hbm.at[pl.ds(tr * 8, rows), pl.ds(i0p, last_w)],
                )

        for k in range(max(0, kF - 1), kF):
            for d in out_descs[k]:
                d.wait()

    return onehot_kernel


def kernel(im_inds, obj_fmaps, obj_labels, rel_inds):
    n = obj_labels.shape[0]
    call = _onehot_call(n)
    return call(obj_labels).T
